# Initial kernel scaffold; baseline (speedup 1.0000x reference)
import jax
import jax.numpy as jnp
from jax.experimental import pallas as pl


def kernel(subgraph_x, subgraph_pos, graph_pos):
    raise NotImplementedError("write your pallas kernel here")


# speedup vs baseline: 3.9302x; 3.9302x over previous
import functools

import jax
import jax.numpy as jnp
from jax import lax
from jax.experimental import pallas as pl
from jax.experimental.pallas import tpu as pltpu
from jax.experimental.pallas import tpu_sc as plsc


_QB = 1000
_S = 10000
_SP = 10240
_SC = 512
_BIGI = 1 << 30


def _nn_body(qx_ref, qy_ref, qz_ref, sx_ref, sy_ref, sz_ref, o_ref):
    qx = qx_ref[...]
    qy = qy_ref[...]
    qz = qz_ref[...]

    def chunk(c, carry):
        bestd, besti = carry
        base = pl.multiple_of(c * _SC, _SC)
        dx = qx - sx_ref[:, pl.ds(base, _SC)]
        dy = qy - sy_ref[:, pl.ds(base, _SC)]
        dz = qz - sz_ref[:, pl.ds(base, _SC)]
        d = dx * dx + dy * dy + dz * dz
        ii = lax.broadcasted_iota(jnp.int32, (_QB, _SC), 1) + base
        m = d < bestd
        return jnp.where(m, d, bestd), jnp.where(m, ii, besti)

    bestd0 = jnp.full((_QB, _SC), jnp.inf, dtype=jnp.float32)
    besti0 = jnp.zeros((_QB, _SC), dtype=jnp.int32)
    bestd, besti = lax.fori_loop(0, _SP // _SC, chunk, (bestd0, besti0))
    minv = jnp.min(bestd, axis=1, keepdims=True)
    cand = jnp.where(bestd == minv, besti, jnp.int32(_BIGI))
    o_ref[...] = jnp.min(cand, axis=1, keepdims=True)


def _nn_idx_tc(graph_pos, sub_pos):
    n = graph_pos.shape[0]
    qx = graph_pos[:, 0:1]
    qy = graph_pos[:, 1:2]
    qz = graph_pos[:, 2:3]
    spt = jnp.pad(sub_pos.T, ((0, 0), (0, _SP - _S)), constant_values=1e9)
    sx, sy, sz = spt[0:1], spt[1:2], spt[2:3]
    q_spec = pl.BlockSpec((_QB, 1), lambda i: (i, 0))
    s_spec = pl.BlockSpec((1, _SP), lambda i: (0, 0))
    idx = pl.pallas_call(
        _nn_body,
        grid=(n // _QB,),
        in_specs=[q_spec, q_spec, q_spec, s_spec, s_spec, s_spec],
        out_specs=pl.BlockSpec((_QB, 1), lambda i: (i, 0)),
        out_shape=jax.ShapeDtypeStruct((n, 1), jnp.int32),
    )(qx, qy, qz, sx, sy, sz)
    return idx.reshape(-1)


_NC = 2
_NS = 16
_NW = _NC * _NS
_CH = 128
_KPW = 25
_BPAD = _NW * _KPW * _CH


def _gather_sc(table, idx_pad):
    mesh = plsc.VectorSubcoreMesh(core_axis_name="c", subcore_axis_name="s")
    bpw = _KPW * _CH

    @functools.partial(
        pl.kernel,
        mesh=mesh,
        out_type=jax.ShapeDtypeStruct((_BPAD, 128), jnp.float32),
        scratch_types=[
            pltpu.VMEM((bpw,), jnp.int32),
            pltpu.VMEM((_CH, 128), jnp.float32),
            pltpu.SemaphoreType.DMA,
        ],
    )
    def k(idx_hbm, table_hbm, out_hbm, idx_v, buf, sem):
        wid = lax.axis_index("s") * _NC + lax.axis_index("c")
        base = pl.multiple_of(wid * bpw, _CH)
        pltpu.sync_copy(idx_hbm.at[pl.ds(base, bpw)], idx_v)

        def chunk(kk, carry):
            off = pl.multiple_of(kk * _CH, _CH)
            pltpu.async_copy(
                table_hbm.at[idx_v.at[pl.ds(off, _CH)]], buf, sem
            ).wait()
            pltpu.sync_copy(buf, out_hbm.at[pl.ds(base + off, _CH)])
            return carry

        lax.fori_loop(0, _KPW, chunk, 0)

    return k(idx_pad, table)


def kernel(subgraph_x, subgraph_pos, graph_pos):
    idx = _nn_idx_tc(graph_pos, subgraph_pos)
    idx_pad = jnp.pad(idx, (0, _BPAD - idx.shape[0]))
    feat = _gather_sc(subgraph_x, idx_pad)
    return feat[: idx.shape[0]]


# speedup vs baseline: 5.5560x; 1.4137x over previous
import functools

import jax
import jax.numpy as jnp
from jax import lax
from jax.experimental import pallas as pl
from jax.experimental.pallas import tpu as pltpu
from jax.experimental.pallas import tpu_sc as plsc


_S = 10000
_QL = 128
_UN = 10
_BIGI = 1 << 30


_ACC = 5


def _nn_body(q_ref, sx_ref, sy_ref, sz_ref, o_ref):
    q = q_ref[...].reshape(3, _QL)
    qx = jnp.broadcast_to(q[0:1], (8, _QL))
    qy = jnp.broadcast_to(q[1:2], (8, _QL))
    qz = jnp.broadcast_to(q[2:3], (8, _QL))
    iot = lax.broadcasted_iota(jnp.int32, (8, _QL), 0)

    def chunk(i, carry):
        ds, js = list(carry[0]), list(carry[1])
        for u in range(_UN):
            c = i * _UN + u
            a = u % _ACC
            base = pl.multiple_of(c * 8, 8)
            dx = qx - sx_ref[pl.ds(base, 8), :]
            dy = qy - sy_ref[pl.ds(base, 8), :]
            dz = qz - sz_ref[pl.ds(base, 8), :]
            d = dx * dx + dy * dy + dz * dz
            ii = iot + base
            m = d < ds[a]
            ds[a] = jnp.where(m, d, ds[a])
            js[a] = jnp.where(m, ii, js[a])
        return tuple(ds), tuple(js)

    inf = jnp.full((8, _QL), jnp.inf, dtype=jnp.float32)
    zero = jnp.zeros((8, _QL), dtype=jnp.int32)
    carry0 = ((inf,) * _ACC, (zero,) * _ACC)
    ds, js = lax.fori_loop(0, _S // (8 * _UN), chunk, carry0)
    bestd, besti = ds[0], js[0]
    for a in range(1, _ACC):
        m = (ds[a] < bestd) | ((ds[a] == bestd) & (js[a] < besti))
        bestd = jnp.where(m, ds[a], bestd)
        besti = jnp.where(m, js[a], besti)
    minv = jnp.min(bestd, axis=0, keepdims=True)
    cand = jnp.where(bestd == minv, besti, jnp.int32(_BIGI))
    o_ref[...] = jnp.min(cand, axis=0, keepdims=True).reshape(1, 1, _QL)


def _nn_idx_tc(graph_pos, sub_pos):
    n = graph_pos.shape[0]
    ng = -(-n // _QL)
    npad = ng * _QL
    q = jnp.pad(graph_pos, ((0, npad - n), (0, 0))).reshape(ng, _QL, 3)
    q = jnp.transpose(q, (0, 2, 1))
    sb = jnp.broadcast_to(sub_pos.T[:, :, None], (3, _S, _QL))
    idx = pl.pallas_call(
        _nn_body,
        grid=(ng,),
        in_specs=[
            pl.BlockSpec((1, 3, _QL), lambda i: (i, 0, 0)),
            pl.BlockSpec((_S, _QL), lambda i: (0, 0)),
            pl.BlockSpec((_S, _QL), lambda i: (0, 0)),
            pl.BlockSpec((_S, _QL), lambda i: (0, 0)),
        ],
        out_specs=pl.BlockSpec((1, 1, _QL), lambda i: (i, 0, 0)),
        out_shape=jax.ShapeDtypeStruct((ng, 1, _QL), jnp.int32),
    )(q, sb[0], sb[1], sb[2])
    return idx.reshape(-1)[:n]


_NC = 2
_NS = 16
_NW = _NC * _NS
_CH = 128
_KPW = 25
_BPAD = _NW * _KPW * _CH


def _gather_sc(table, idx_pad):
    mesh = plsc.VectorSubcoreMesh(core_axis_name="c", subcore_axis_name="s")
    bpw = _KPW * _CH

    @functools.partial(
        pl.kernel,
        mesh=mesh,
        out_type=jax.ShapeDtypeStruct((_BPAD, 128), jnp.float32),
        scratch_types=[
            pltpu.VMEM((bpw,), jnp.int32),
            pltpu.VMEM((_CH, 128), jnp.float32),
            pltpu.SemaphoreType.DMA,
        ],
    )
    def k(idx_hbm, table_hbm, out_hbm, idx_v, buf, sem):
        wid = lax.axis_index("s") * _NC + lax.axis_index("c")
        base = pl.multiple_of(wid * bpw, _CH)
        pltpu.sync_copy(idx_hbm.at[pl.ds(base, bpw)], idx_v)

        def chunk(kk, carry):
            off = pl.multiple_of(kk * _CH, _CH)
            pltpu.async_copy(
                table_hbm.at[idx_v.at[pl.ds(off, _CH)]], buf, sem
            ).wait()
            pltpu.sync_copy(buf, out_hbm.at[pl.ds(base + off, _CH)])
            return carry

        lax.fori_loop(0, _KPW, chunk, 0)

    return k(idx_pad, table)


def kernel(subgraph_x, subgraph_pos, graph_pos):
    idx = _nn_idx_tc(graph_pos, subgraph_pos)
    idx_pad = jnp.pad(idx, (0, _BPAD - idx.shape[0]))
    feat = _gather_sc(subgraph_x, idx_pad)
    return feat[: idx.shape[0]]
